# baseline (speedup 1.0000x reference)
import functools
import jax
import jax.numpy as jnp
from jax import lax
from jax.experimental import pallas as pl
from jax.experimental.pallas import tpu as pltpu
from jax.experimental.pallas import tpu_sc as plsc

N = 10000
E = 320000
D = 128
T = 4
TAU = 2.0
V_TH = 1.0

NP = 10240
NC = 2
NS = 16
NW = NC * NS
CHUNK = 100
CPW = E // (NW * CHUNK)
PH = 4
CPP = CPW // PH
ROWS_PW = NP // NS
ZROWS = 80

_mesh = plsc.VectorSubcoreMesh(core_axis_name="c", subcore_axis_name="s")


@functools.partial(
    pl.kernel,
    out_type=jax.ShapeDtypeStruct((NC, NP), jnp.float32),
    mesh=_mesh,
    scratch_types=[
        pltpu.VMEM((CPW, 1, CHUNK), jnp.int32),
        pltpu.VMEM((D,), jnp.float32),
        pltpu.VMEM((ROWS_PW,), jnp.float32),
        pltpu.VMEM_SHARED((NP,), jnp.float32),
    ],
)
def _deg_kernel(dst_hbm, ones_hbm, zeros_hbm, out_hbm, didx_v, ones_v, zb_v, dacc):
    cid = lax.axis_index("c")
    sid = lax.axis_index("s")
    wid = sid * NC + cid
    pltpu.sync_copy(ones_hbm, ones_v)
    pltpu.sync_copy(zeros_hbm, zb_v)
    pltpu.sync_copy(dst_hbm.at[pl.ds(wid * CPW, CPW)], didx_v)
    pltpu.sync_copy(zb_v, dacc.at[pl.ds(sid * ROWS_PW, ROWS_PW)])
    plsc.subcore_barrier()

    def body(j, carry):
        pltpu.sync_copy(ones_v.at[pl.ds(0, CHUNK)], dacc.at[didx_v.at[j, 0]],
                        add=True)
        return carry

    lax.fori_loop(0, CPW, body, 0)
    plsc.subcore_barrier()
    pltpu.sync_copy(dacc.at[pl.ds(sid * ROWS_PW, ROWS_PW)], zb_v)
    pltpu.sync_copy(zb_v, out_hbm.at[cid, pl.ds(sid * ROWS_PW, ROWS_PW)])


@functools.partial(
    pl.kernel,
    out_type=jax.ShapeDtypeStruct((NC, NP, D), jnp.float32),
    mesh=_mesh,
    scratch_types=[
        pltpu.VMEM((CPP, 1, CHUNK), jnp.int32),
        pltpu.VMEM((CPP, 1, CHUNK), jnp.int32),
        pltpu.VMEM((CHUNK, D), jnp.float32),
        pltpu.VMEM((CHUNK, D), jnp.float32),
        pltpu.VMEM((CHUNK, D), jnp.float32),
        pltpu.VMEM_SHARED((NP, D), jnp.float32),
        pltpu.SemaphoreType.DMA,
        pltpu.SemaphoreType.DMA,
        pltpu.SemaphoreType.DMA,
        pltpu.SemaphoreType.DMA,
        pltpu.SemaphoreType.DMA,
        pltpu.SemaphoreType.DMA,
    ],
)
def _edge_kernel(h_hbm, src_hbm, dst_hbm, zeros_hbm, out_hbm,
                 sidx_v, didx_v, rows_0, rows_1, rows_2, acc,
                 sem_g0, sem_g1, sem_g2, sem_s0, sem_s1, sem_s2):
    cid = lax.axis_index("c")
    sid = lax.axis_index("s")
    wid = sid * NC + cid
    row0 = sid * ROWS_PW
    pltpu.sync_copy(zeros_hbm, rows_0)
    for k in range(ROWS_PW // ZROWS):
        pltpu.sync_copy(rows_0.at[pl.ds(0, ZROWS)],
                        acc.at[pl.ds(row0 + k * ZROWS, ZROWS)])
    plsc.subcore_barrier()

    bufs = (rows_0, rows_1, rows_2)
    gsems = (sem_g0, sem_g1, sem_g2)
    ssems = (sem_s0, sem_s1, sem_s2)

    def gather(j, p):
        return pltpu.async_copy(h_hbm.at[sidx_v.at[j, 0]], bufs[p], gsems[p])

    def scatter(j, p):
        return pltpu.async_copy(bufs[p], acc.at[didx_v.at[j, 0]], ssems[p],
                                add=True)

    def drain(p, sem3):
        pltpu.make_async_copy(zeros_hbm, bufs[p], sem3[p]).wait()

    for ph in range(PH):
        pltpu.sync_copy(src_hbm.at[pl.ds(wid * CPW + ph * CPP, CPP)], sidx_v)
        pltpu.sync_copy(dst_hbm.at[pl.ds(wid * CPW + ph * CPP, CPP)], didx_v)

        gather(0, 0).wait()
        scatter(0, 0)
        gather(1, 1).wait()
        scatter(1, 1)
        gather(2, 2).wait()
        scatter(2, 2)
        drain(0, ssems)
        gather(3, 0)

        def step(c, p):
            drain(p, gsems)
            scatter(c, p)
            q = (p + 1) % 3
            drain(q, ssems)
            gather(c + 1, q)

        def body(g, carry):
            a = 3 * g
            step(a, 0)
            step(a + 1, 1)
            step(a + 2, 2)
            return carry

        lax.fori_loop(1, CPP // 3, body, 0)
        drain(0, gsems)
        scatter(CPP - 1, 0)
        drain(1, ssems)
        drain(2, ssems)
        drain(0, ssems)

    plsc.subcore_barrier()
    for k in range(ROWS_PW // ZROWS):
        pltpu.sync_copy(acc.at[pl.ds(row0 + k * ZROWS, ZROWS)],
                        out_hbm.at[cid, pl.ds(row0 + k * ZROWS, ZROWS)])


BLK = 1280
GRID = NP // BLK


def _dinv_of(degt_blk):
    deg = degt_blk[:, 0:1] + degt_blk[:, 1:2]
    deg = jnp.maximum(deg, 1.0)
    return lax.rsqrt(deg)


def _k1_body(x_ref, w1_ref, degt_ref, h1p_ref):
    dinv = _dinv_of(degt_ref[...])
    h = jnp.dot(x_ref[...], w1_ref[...], preferred_element_type=jnp.float32)
    h1p_ref[...] = h * dinv


def _k2_body(aggp_ref, degt_ref, b1_ref, w2_ref, h2p_ref):
    dinv = _dinv_of(degt_ref[...])
    s = aggp_ref[0] + aggp_ref[1]
    h1 = jnp.maximum(s * dinv + b1_ref[...], 0.0)
    h2 = jnp.dot(h1, w2_ref[...], preferred_element_type=jnp.float32)
    h2p_ref[...] = h2 * dinv


def _k3_body(aggp_ref, degt_ref, b2_ref, out_ref):
    dinv = _dinv_of(degt_ref[...])
    s = aggp_ref[0] + aggp_ref[1]
    y = s * dinv + b2_ref[...]
    v = jnp.zeros_like(y)
    for t in range(T):
        v = v + (y - v) / TAU
        spike = (v >= V_TH).astype(jnp.float32)
        out_ref[t] = spike
        v = v * (1.0 - spike)


def _tc_k1(x, w1, degt):
    return pl.pallas_call(
        _k1_body,
        grid=(GRID,),
        in_specs=[
            pl.BlockSpec((BLK, D), lambda i: (i, 0)),
            pl.BlockSpec((D, D), lambda i: (0, 0)),
            pl.BlockSpec((BLK, NC), lambda i: (i, 0)),
        ],
        out_specs=pl.BlockSpec((BLK, D), lambda i: (i, 0)),
        out_shape=jax.ShapeDtypeStruct((N, D), jnp.float32),
    )(x, w1, degt)


def _tc_k2(aggp, degt, b1, w2):
    return pl.pallas_call(
        _k2_body,
        grid=(GRID,),
        in_specs=[
            pl.BlockSpec((NC, BLK, D), lambda i: (0, i, 0)),
            pl.BlockSpec((BLK, NC), lambda i: (i, 0)),
            pl.BlockSpec((1, D), lambda i: (0, 0)),
            pl.BlockSpec((D, D), lambda i: (0, 0)),
        ],
        out_specs=pl.BlockSpec((BLK, D), lambda i: (i, 0)),
        out_shape=jax.ShapeDtypeStruct((N, D), jnp.float32),
    )(aggp, degt, b1, w2)


def _tc_k3(aggp, degt, b2):
    return pl.pallas_call(
        _k3_body,
        grid=(GRID,),
        in_specs=[
            pl.BlockSpec((NC, BLK, D), lambda i: (0, i, 0)),
            pl.BlockSpec((BLK, NC), lambda i: (i, 0)),
            pl.BlockSpec((1, D), lambda i: (0, 0)),
        ],
        out_specs=pl.BlockSpec((T, BLK, D), lambda i: (0, i, 0)),
        out_shape=jax.ShapeDtypeStruct((T, N, D), jnp.float32),
    )(aggp, degt, b2)


@jax.jit
def kernel(x, edge_index, W1, b1, W2, b2):
    src = edge_index[0].astype(jnp.int32).reshape(NW * CPW, 1, CHUNK)
    dst = edge_index[1].astype(jnp.int32).reshape(NW * CPW, 1, CHUNK)
    ones_h = jnp.ones((D,), jnp.float32)
    zeros1 = jnp.zeros((ROWS_PW,), jnp.float32)
    zeros2 = jnp.zeros((CHUNK, D), jnp.float32)

    degp = _deg_kernel(dst, ones_h, zeros1)
    degt = jnp.swapaxes(degp, 0, 1)

    h1p = _tc_k1(x, W1, degt)
    agg1p = _edge_kernel(h1p, src, dst, zeros2)
    h2p = _tc_k2(agg1p, degt, b1.reshape(1, D), W2)
    agg2p = _edge_kernel(h2p, src, dst, zeros2)
    out = _tc_k3(agg2p, degt, b2.reshape(1, D))
    return out


# speedup vs baseline: 1.0359x; 1.0359x over previous
import functools
import jax
import jax.numpy as jnp
from jax import lax
from jax.experimental import pallas as pl
from jax.experimental.pallas import tpu as pltpu
from jax.experimental.pallas import tpu_sc as plsc

N = 10000
E = 320000
D = 128
T = 4
TAU = 2.0
V_TH = 1.0

NP = 10240
NC = 2
NS = 16
NW = NC * NS
CHUNK = 100
CPW = E // (NW * CHUNK)
PH = 2
CPP = CPW // PH
ROWS_PW = NP // NS
ZROWS = 80

_mesh = plsc.VectorSubcoreMesh(core_axis_name="c", subcore_axis_name="s")


@functools.partial(
    pl.kernel,
    out_type=jax.ShapeDtypeStruct((NC, NP), jnp.float32),
    mesh=_mesh,
    scratch_types=[
        pltpu.VMEM((CPW, 1, CHUNK), jnp.int32),
        pltpu.VMEM((D,), jnp.float32),
        pltpu.VMEM((ROWS_PW,), jnp.float32),
        pltpu.VMEM_SHARED((NP,), jnp.float32),
        pltpu.SemaphoreType.DMA,
        pltpu.SemaphoreType.DMA,
    ],
)
def _deg_kernel(dst_hbm, ones_hbm, zeros_hbm, out_hbm,
                didx_v, ones_v, zb_v, dacc, sem0, sem1):
    cid = lax.axis_index("c")
    sid = lax.axis_index("s")
    wid = sid * NC + cid
    pltpu.sync_copy(ones_hbm, ones_v)
    pltpu.sync_copy(zeros_hbm, zb_v)
    pltpu.sync_copy(dst_hbm.at[pl.ds(wid * CPW, CPW)], didx_v)
    pltpu.sync_copy(zb_v, dacc.at[pl.ds(sid * ROWS_PW, ROWS_PW)])
    plsc.subcore_barrier()

    def body(g, carry):
        d0 = pltpu.async_copy(ones_v.at[pl.ds(0, CHUNK)],
                              dacc.at[didx_v.at[2 * g, 0]], sem0, add=True)
        d1 = pltpu.async_copy(ones_v.at[pl.ds(0, CHUNK)],
                              dacc.at[didx_v.at[2 * g + 1, 0]], sem1, add=True)
        d0.wait()
        d1.wait()
        return carry

    lax.fori_loop(0, CPW // 2, body, 0)
    plsc.subcore_barrier()
    pltpu.sync_copy(dacc.at[pl.ds(sid * ROWS_PW, ROWS_PW)], zb_v)
    pltpu.sync_copy(zb_v, out_hbm.at[cid, pl.ds(sid * ROWS_PW, ROWS_PW)])


@functools.partial(
    pl.kernel,
    out_type=jax.ShapeDtypeStruct((NC, NP, D), jnp.float32),
    mesh=_mesh,
    scratch_types=[
        pltpu.VMEM((CPP, 1, CHUNK), jnp.int32),
        pltpu.VMEM((CPP, 1, CHUNK), jnp.int32),
        pltpu.VMEM((CHUNK, D), jnp.float32),
        pltpu.VMEM((CHUNK, D), jnp.float32),
        pltpu.VMEM_SHARED((NP, D), jnp.float32),
        pltpu.SemaphoreType.DMA,
        pltpu.SemaphoreType.DMA,
        pltpu.SemaphoreType.DMA,
        pltpu.SemaphoreType.DMA,
    ],
)
def _edge_kernel(h_hbm, src_hbm, dst_hbm, zeros_hbm, out_hbm,
                 sidx_v, didx_v, rows_a, rows_b, acc,
                 sem_ga, sem_gb, sem_sa, sem_sb):
    cid = lax.axis_index("c")
    sid = lax.axis_index("s")
    wid = sid * NC + cid
    row0 = sid * ROWS_PW
    pltpu.sync_copy(zeros_hbm, rows_a)
    for k in range(ROWS_PW // ZROWS):
        pltpu.sync_copy(rows_a.at[pl.ds(0, ZROWS)],
                        acc.at[pl.ds(row0 + k * ZROWS, ZROWS)])
    plsc.subcore_barrier()

    def gather(j, buf, sem):
        return pltpu.async_copy(h_hbm.at[sidx_v.at[j, 0]], buf, sem)

    def scatter(j, buf, sem):
        return pltpu.async_copy(buf, acc.at[didx_v.at[j, 0]], sem, add=True)

    def drain(buf, sem):
        pltpu.make_async_copy(zeros_hbm, buf, sem).wait()

    for ph in range(PH):
        pltpu.sync_copy(src_hbm.at[pl.ds(wid * CPW + ph * CPP, CPP)], sidx_v)
        pltpu.sync_copy(dst_hbm.at[pl.ds(wid * CPW + ph * CPP, CPP)], didx_v)

        gather(0, rows_a, sem_ga).wait()
        scatter(0, rows_a, sem_sa)
        gather(1, rows_b, sem_gb).wait()
        scatter(1, rows_b, sem_sb)
        drain(rows_a, sem_sa)
        gather(2, rows_a, sem_ga)

        def body(g, carry):
            a = 2 * g
            drain(rows_a, sem_ga)
            scatter(a, rows_a, sem_sa)
            drain(rows_b, sem_sb)
            gather(a + 1, rows_b, sem_gb)
            drain(rows_b, sem_gb)
            scatter(a + 1, rows_b, sem_sb)
            drain(rows_a, sem_sa)
            gather(a + 2, rows_a, sem_ga)
            return carry

        lax.fori_loop(1, CPP // 2 - 1, body, 0)
        drain(rows_a, sem_ga)
        scatter(CPP - 2, rows_a, sem_sa)
        drain(rows_b, sem_sb)
        gather(CPP - 1, rows_b, sem_gb).wait()
        scatter(CPP - 1, rows_b, sem_sb)
        drain(rows_a, sem_sa)
        drain(rows_b, sem_sb)

    plsc.subcore_barrier()
    for k in range(ROWS_PW // ZROWS):
        pltpu.sync_copy(acc.at[pl.ds(row0 + k * ZROWS, ZROWS)],
                        out_hbm.at[cid, pl.ds(row0 + k * ZROWS, ZROWS)])


BLK = 1280
GRID = NP // BLK


def _dinv_of(degt_blk):
    deg = degt_blk[:, 0:1] + degt_blk[:, 1:2]
    deg = jnp.maximum(deg, 1.0)
    return lax.rsqrt(deg)


def _k1_body(x_ref, w1_ref, degt_ref, h1p_ref):
    dinv = _dinv_of(degt_ref[...])
    h = jnp.dot(x_ref[...], w1_ref[...], preferred_element_type=jnp.float32)
    h1p_ref[...] = h * dinv


def _k2_body(aggp_ref, degt_ref, b1_ref, w2_ref, h2p_ref):
    dinv = _dinv_of(degt_ref[...])
    s = aggp_ref[0] + aggp_ref[1]
    h1 = jnp.maximum(s * dinv + b1_ref[...], 0.0)
    h2 = jnp.dot(h1, w2_ref[...], preferred_element_type=jnp.float32)
    h2p_ref[...] = h2 * dinv


def _k3_body(aggp_ref, degt_ref, b2_ref, out_ref):
    dinv = _dinv_of(degt_ref[...])
    s = aggp_ref[0] + aggp_ref[1]
    y = s * dinv + b2_ref[...]
    v = jnp.zeros_like(y)
    for t in range(T):
        v = v + (y - v) / TAU
        spike = (v >= V_TH).astype(jnp.float32)
        out_ref[t] = spike
        v = v * (1.0 - spike)


def _tc_k1(x, w1, degt):
    return pl.pallas_call(
        _k1_body,
        grid=(GRID,),
        in_specs=[
            pl.BlockSpec((BLK, D), lambda i: (i, 0)),
            pl.BlockSpec((D, D), lambda i: (0, 0)),
            pl.BlockSpec((BLK, NC), lambda i: (i, 0)),
        ],
        out_specs=pl.BlockSpec((BLK, D), lambda i: (i, 0)),
        out_shape=jax.ShapeDtypeStruct((N, D), jnp.float32),
    )(x, w1, degt)


def _tc_k2(aggp, degt, b1, w2):
    return pl.pallas_call(
        _k2_body,
        grid=(GRID,),
        in_specs=[
            pl.BlockSpec((NC, BLK, D), lambda i: (0, i, 0)),
            pl.BlockSpec((BLK, NC), lambda i: (i, 0)),
            pl.BlockSpec((1, D), lambda i: (0, 0)),
            pl.BlockSpec((D, D), lambda i: (0, 0)),
        ],
        out_specs=pl.BlockSpec((BLK, D), lambda i: (i, 0)),
        out_shape=jax.ShapeDtypeStruct((N, D), jnp.float32),
    )(aggp, degt, b1, w2)


def _tc_k3(aggp, degt, b2):
    return pl.pallas_call(
        _k3_body,
        grid=(GRID,),
        in_specs=[
            pl.BlockSpec((NC, BLK, D), lambda i: (0, i, 0)),
            pl.BlockSpec((BLK, NC), lambda i: (i, 0)),
            pl.BlockSpec((1, D), lambda i: (0, 0)),
        ],
        out_specs=pl.BlockSpec((T, BLK, D), lambda i: (0, i, 0)),
        out_shape=jax.ShapeDtypeStruct((T, N, D), jnp.float32),
    )(aggp, degt, b2)


@jax.jit
def kernel(x, edge_index, W1, b1, W2, b2):
    src = edge_index[0].astype(jnp.int32).reshape(NW * CPW, 1, CHUNK)
    dst = edge_index[1].astype(jnp.int32).reshape(NW * CPW, 1, CHUNK)
    ones_h = jnp.ones((D,), jnp.float32)
    zeros1 = jnp.zeros((ROWS_PW,), jnp.float32)
    zeros2 = jnp.zeros((CHUNK, D), jnp.float32)

    degp = _deg_kernel(dst, ones_h, zeros1)
    degt = jnp.swapaxes(degp, 0, 1)

    h1p = _tc_k1(x, W1, degt)
    agg1p = _edge_kernel(h1p, src, dst, zeros2)
    h2p = _tc_k2(agg1p, degt, b1.reshape(1, D), W2)
    agg2p = _edge_kernel(h2p, src, dst, zeros2)
    out = _tc_k3(agg2p, degt, b2.reshape(1, D))
    return out


# speedup vs baseline: 1.0390x; 1.0030x over previous
import functools
import jax
import jax.numpy as jnp
from jax import lax
from jax.experimental import pallas as pl
from jax.experimental.pallas import tpu as pltpu
from jax.experimental.pallas import tpu_sc as plsc

N = 10000
E = 320000
D = 128
T = 4
TAU = 2.0
V_TH = 1.0

NP = 10240
NC = 2
NS = 16
NW = NC * NS
CHUNK = 100
CPW = E // (NW * CHUNK)
PH = 2
CPP = CPW // PH
ROWS_PW = NP // NS
ZROWS = 80

_mesh = plsc.VectorSubcoreMesh(core_axis_name="c", subcore_axis_name="s")


@functools.partial(
    pl.kernel,
    out_type=jax.ShapeDtypeStruct((NC, NP), jnp.float32),
    mesh=_mesh,
    scratch_types=[
        pltpu.VMEM((CPW, 1, CHUNK), jnp.int32),
        pltpu.VMEM((D,), jnp.float32),
        pltpu.VMEM((ROWS_PW,), jnp.float32),
        pltpu.VMEM_SHARED((NP,), jnp.float32),
        pltpu.SemaphoreType.DMA,
        pltpu.SemaphoreType.DMA,
        pltpu.SemaphoreType.DMA,
        pltpu.SemaphoreType.DMA,
    ],
)
def _deg_kernel(dst_hbm, ones_hbm, zeros_hbm, out_hbm,
                didx_v, ones_v, zb_v, dacc, sem0, sem1, sem2, sem3):
    cid = lax.axis_index("c")
    sid = lax.axis_index("s")
    wid = sid * NC + cid
    pltpu.sync_copy(ones_hbm, ones_v)
    pltpu.sync_copy(zeros_hbm, zb_v)
    pltpu.sync_copy(dst_hbm.at[pl.ds(wid * CPW, CPW)], didx_v)
    pltpu.sync_copy(zb_v, dacc.at[pl.ds(sid * ROWS_PW, ROWS_PW)])
    plsc.subcore_barrier()

    def body(g, carry):
        sems = (sem0, sem1, sem2, sem3)
        ds = [pltpu.async_copy(ones_v.at[pl.ds(0, CHUNK)],
                               dacc.at[didx_v.at[4 * g + q, 0]], sems[q],
                               add=True)
              for q in range(4)]
        for d in ds:
            d.wait()
        return carry

    lax.fori_loop(0, CPW // 4, body, 0)
    plsc.subcore_barrier()
    pltpu.sync_copy(dacc.at[pl.ds(sid * ROWS_PW, ROWS_PW)], zb_v)
    pltpu.sync_copy(zb_v, out_hbm.at[cid, pl.ds(sid * ROWS_PW, ROWS_PW)])


@functools.partial(
    pl.kernel,
    out_type=jax.ShapeDtypeStruct((NC, NP, D), jnp.float32),
    mesh=_mesh,
    scratch_types=[
        pltpu.VMEM((CPP, 1, CHUNK), jnp.int32),
        pltpu.VMEM((CPP, 1, CHUNK), jnp.int32),
        pltpu.VMEM((CHUNK, D), jnp.float32),
        pltpu.VMEM((CHUNK, D), jnp.float32),
        pltpu.VMEM_SHARED((NP, D), jnp.float32),
        pltpu.SemaphoreType.DMA,
        pltpu.SemaphoreType.DMA,
        pltpu.SemaphoreType.DMA,
        pltpu.SemaphoreType.DMA,
    ],
)
def _edge_kernel(h_hbm, src_hbm, dst_hbm, zeros_hbm, out_hbm,
                 sidx_v, didx_v, rows_a, rows_b, acc,
                 sem_ga, sem_gb, sem_sa, sem_sb):
    cid = lax.axis_index("c")
    sid = lax.axis_index("s")
    wid = sid * NC + cid
    row0 = sid * ROWS_PW
    pltpu.sync_copy(zeros_hbm, rows_a)
    for k in range(ROWS_PW // ZROWS):
        pltpu.sync_copy(rows_a.at[pl.ds(0, ZROWS)],
                        acc.at[pl.ds(row0 + k * ZROWS, ZROWS)])
    plsc.subcore_barrier()

    def gather(j, buf, sem):
        return pltpu.async_copy(h_hbm.at[sidx_v.at[j, 0]], buf, sem)

    def scatter(j, buf, sem):
        return pltpu.async_copy(buf, acc.at[didx_v.at[j, 0]], sem, add=True)

    def drain(buf, sem):
        pltpu.make_async_copy(zeros_hbm, buf, sem).wait()

    for ph in range(PH):
        pltpu.sync_copy(src_hbm.at[pl.ds(wid * CPW + ph * CPP, CPP)], sidx_v)
        pltpu.sync_copy(dst_hbm.at[pl.ds(wid * CPW + ph * CPP, CPP)], didx_v)

        gather(0, rows_a, sem_ga).wait()
        scatter(0, rows_a, sem_sa)
        gather(1, rows_b, sem_gb).wait()
        scatter(1, rows_b, sem_sb)
        drain(rows_a, sem_sa)
        gather(2, rows_a, sem_ga)

        def body(g, carry):
            a = 2 * g
            drain(rows_a, sem_ga)
            scatter(a, rows_a, sem_sa)
            drain(rows_b, sem_sb)
            gather(a + 1, rows_b, sem_gb)
            drain(rows_b, sem_gb)
            scatter(a + 1, rows_b, sem_sb)
            drain(rows_a, sem_sa)
            gather(a + 2, rows_a, sem_ga)
            return carry

        lax.fori_loop(1, CPP // 2 - 1, body, 0)
        drain(rows_a, sem_ga)
        scatter(CPP - 2, rows_a, sem_sa)
        drain(rows_b, sem_sb)
        gather(CPP - 1, rows_b, sem_gb).wait()
        scatter(CPP - 1, rows_b, sem_sb)
        drain(rows_a, sem_sa)
        drain(rows_b, sem_sb)

    plsc.subcore_barrier()
    for k in range(ROWS_PW // ZROWS):
        pltpu.sync_copy(acc.at[pl.ds(row0 + k * ZROWS, ZROWS)],
                        out_hbm.at[cid, pl.ds(row0 + k * ZROWS, ZROWS)])


BLK = 1280
GRID = NP // BLK


def _dinv_of(degt_blk):
    deg = degt_blk[:, 0:1] + degt_blk[:, 1:2]
    deg = jnp.maximum(deg, 1.0)
    return lax.rsqrt(deg)


def _k1_body(x_ref, w1_ref, degt_ref, h1p_ref):
    dinv = _dinv_of(degt_ref[...])
    h = jnp.dot(x_ref[...], w1_ref[...], preferred_element_type=jnp.float32)
    h1p_ref[...] = h * dinv


def _k2_body(aggp_ref, degt_ref, b1_ref, w2_ref, h2p_ref):
    dinv = _dinv_of(degt_ref[...])
    s = aggp_ref[0] + aggp_ref[1]
    h1 = jnp.maximum(s * dinv + b1_ref[...], 0.0)
    h2 = jnp.dot(h1, w2_ref[...], preferred_element_type=jnp.float32)
    h2p_ref[...] = h2 * dinv


def _k3_body(aggp_ref, degt_ref, b2_ref, out_ref):
    dinv = _dinv_of(degt_ref[...])
    s = aggp_ref[0] + aggp_ref[1]
    y = s * dinv + b2_ref[...]
    v = jnp.zeros_like(y)
    for t in range(T):
        v = v + (y - v) / TAU
        spike = (v >= V_TH).astype(jnp.float32)
        out_ref[t] = spike
        v = v * (1.0 - spike)


def _tc_k1(x, w1, degt):
    return pl.pallas_call(
        _k1_body,
        grid=(GRID,),
        in_specs=[
            pl.BlockSpec((BLK, D), lambda i: (i, 0)),
            pl.BlockSpec((D, D), lambda i: (0, 0)),
            pl.BlockSpec((BLK, NC), lambda i: (i, 0)),
        ],
        out_specs=pl.BlockSpec((BLK, D), lambda i: (i, 0)),
        out_shape=jax.ShapeDtypeStruct((N, D), jnp.float32),
    )(x, w1, degt)


def _tc_k2(aggp, degt, b1, w2):
    return pl.pallas_call(
        _k2_body,
        grid=(GRID,),
        in_specs=[
            pl.BlockSpec((NC, BLK, D), lambda i: (0, i, 0)),
            pl.BlockSpec((BLK, NC), lambda i: (i, 0)),
            pl.BlockSpec((1, D), lambda i: (0, 0)),
            pl.BlockSpec((D, D), lambda i: (0, 0)),
        ],
        out_specs=pl.BlockSpec((BLK, D), lambda i: (i, 0)),
        out_shape=jax.ShapeDtypeStruct((N, D), jnp.float32),
    )(aggp, degt, b1, w2)


def _tc_k3(aggp, degt, b2):
    return pl.pallas_call(
        _k3_body,
        grid=(GRID,),
        in_specs=[
            pl.BlockSpec((NC, BLK, D), lambda i: (0, i, 0)),
            pl.BlockSpec((BLK, NC), lambda i: (i, 0)),
            pl.BlockSpec((1, D), lambda i: (0, 0)),
        ],
        out_specs=pl.BlockSpec((T, BLK, D), lambda i: (0, i, 0)),
        out_shape=jax.ShapeDtypeStruct((T, N, D), jnp.float32),
    )(aggp, degt, b2)


@jax.jit
def kernel(x, edge_index, W1, b1, W2, b2):
    src = edge_index[0].astype(jnp.int32).reshape(NW * CPW, 1, CHUNK)
    dst = edge_index[1].astype(jnp.int32).reshape(NW * CPW, 1, CHUNK)
    ones_h = jnp.ones((D,), jnp.float32)
    zeros1 = jnp.zeros((ROWS_PW,), jnp.float32)
    zeros2 = jnp.zeros((CHUNK, D), jnp.float32)

    degp = _deg_kernel(dst, ones_h, zeros1)
    degt = jnp.swapaxes(degp, 0, 1)

    h1p = _tc_k1(x, W1, degt)
    agg1p = _edge_kernel(h1p, src, dst, zeros2)
    h2p = _tc_k2(agg1p, degt, b1.reshape(1, D), W2)
    agg2p = _edge_kernel(h2p, src, dst, zeros2)
    out = _tc_k3(agg2p, degt, b2.reshape(1, D))
    return out
